# baseline (speedup 1.0000x reference)
import functools

import jax
import jax.numpy as jnp
from jax import lax
from jax.experimental import pallas as pl
from jax.experimental.pallas import tpu as pltpu
from jax.experimental.pallas import tpu_sc as plsc

N = 10000
E = 320000
IN_FEAT = 128
OUT_FEAT = 128
NUM_RELS = 8

NUM_CORES = 2
NUM_SUBCORES = 16
NW = NUM_CORES * NUM_SUBCORES
EPW = E // NW
CHUNK = 80
NCHUNK = EPW // CHUNK
NBUF = 3
NPAD = 10240
ROWS_PER_TILE = NPAD // NUM_SUBCORES

DST_BITS = 14

BM = 2000


def _h_body(x_ref, w_ref, b_ref, ei_ref, et_ref, o_ref, op_ref, oz_ref):
    o_ref[0] = (
        jnp.dot(x_ref[...], w_ref[0], preferred_element_type=jnp.float32)
        + b_ref[0]
    )

    @pl.when(pl.program_id(0) == 0)
    def _():
        op_ref[...] = ((et_ref[...] * N + ei_ref[0]) << DST_BITS) | ei_ref[1]
        oz_ref[...] = jnp.zeros((ROWS_PER_TILE, OUT_FEAT), jnp.float32)


def _transform_nodes(x, weight, bias, edge_index, edge_type):
    return pl.pallas_call(
        _h_body,
        grid=(NUM_RELS,),
        in_specs=[
            pl.BlockSpec((N, IN_FEAT), lambda r: (0, 0)),
            pl.BlockSpec((1, IN_FEAT, OUT_FEAT), lambda r: (r, 0, 0)),
            pl.BlockSpec((1, 1, OUT_FEAT), lambda r: (r, 0, 0)),
            pl.BlockSpec((2, E), lambda r: (0, 0)),
            pl.BlockSpec((E,), lambda r: (0,)),
        ],
        out_specs=[
            pl.BlockSpec((1, N, OUT_FEAT), lambda r: (r, 0, 0)),
            pl.BlockSpec((E,), lambda r: (0,)),
            pl.BlockSpec((ROWS_PER_TILE, OUT_FEAT), lambda r: (0, 0)),
        ],
        out_shape=[
            jax.ShapeDtypeStruct((NUM_RELS, N, OUT_FEAT), jnp.float32),
            jax.ShapeDtypeStruct((E,), jnp.int32),
            jax.ShapeDtypeStruct((ROWS_PER_TILE, OUT_FEAT), jnp.float32),
        ],
    )(x, weight, bias.reshape(NUM_RELS, 1, OUT_FEAT), edge_index, edge_type)


def _sc_body(table_hbm, packed_hbm, zeros_hbm, parts_hbm,
             pk_a, pk_b, pk_c, idx_a, dst_a, idx_b, dst_b, idx_c, dst_c,
             rows_a, rows_b, rows_c, acc_sh,
             gsem_a, gsem_b, gsem_c, ssem_a, ssem_b, ssem_c,
             psem_a, psem_b, psem_c):
    c = lax.axis_index("c")
    s = lax.axis_index("s")
    wid = c * NUM_SUBCORES + s
    base = wid * EPW

    def unpack(pbuf, idx_buf, dst_buf, n):
        for i in range(n // 16):
            p = pbuf[pl.ds(i * 16, 16)]
            idx_buf[pl.ds(i * 16, 16)] = p >> DST_BITS
            dst_buf[pl.ds(i * 16, 16)] = p & ((1 << DST_BITS) - 1)

    def wait_rows(buf, sem):
        pltpu.make_async_copy(table_hbm.at[pl.ds(0, CHUNK)], buf, sem).wait()

    def wait_pk(buf, sem):
        pltpu.make_async_copy(packed_hbm.at[pl.ds(0, CHUNK)], buf, sem).wait()

    pk = (pk_a, pk_b, pk_c)
    rows = (rows_a, rows_b, rows_c)
    idxb = (idx_a, idx_b, idx_c)
    dstb = (dst_a, dst_b, dst_c)
    gsem = (gsem_a, gsem_b, gsem_c)
    ssem = (ssem_a, ssem_b, ssem_c)
    psem = (psem_a, psem_b, psem_c)

    for t in range(NBUF):
        pltpu.async_copy(packed_hbm.at[pl.ds(base + t * CHUNK, CHUNK)],
                         pk[t], psem[t])
    pltpu.sync_copy(zeros_hbm, acc_sh.at[pl.ds(s * ROWS_PER_TILE, ROWS_PER_TILE)])
    plsc.subcore_barrier()

    def body(j, carry):
        for t in range(NBUF):
            c_fire = NBUF * j + t

            @pl.when(c_fire < NCHUNK)
            def _():
                @pl.when(c_fire >= NBUF)
                def _():
                    wait_rows(rows[t], ssem[t])

                wait_pk(pk[t], psem[t])
                unpack(pk[t], idxb[t], dstb[t], CHUNK)
                pltpu.async_copy(table_hbm.at[idxb[t]], rows[t], gsem[t])

                @pl.when(c_fire + NBUF < NCHUNK)
                def _():
                    pltpu.async_copy(
                        packed_hbm.at[pl.ds(base + (c_fire + NBUF) * CHUNK, CHUNK)],
                        pk[t], psem[t])

            c_ret = c_fire - (NBUF - 1)

            @pl.when((c_ret >= 0) & (c_ret < NCHUNK))
            def _():
                u = (t + 1) % NBUF
                wait_rows(rows[u], gsem[u])
                pltpu.async_copy(rows[u], acc_sh.at[dstb[u]], ssem[u], add=True)

        return carry

    lax.fori_loop(0, (NCHUNK + 2 * (NBUF - 1) + NBUF - 1) // NBUF, body, 0)
    for t in range(NBUF):
        wait_rows(rows[t], ssem[t])
    plsc.subcore_barrier()
    pltpu.sync_copy(
        acc_sh.at[pl.ds(s * ROWS_PER_TILE, ROWS_PER_TILE)],
        parts_hbm.at[c, pl.ds(s * ROWS_PER_TILE, ROWS_PER_TILE)],
    )


_sc_gather_scatter = functools.partial(
    pl.kernel,
    out_type=jax.ShapeDtypeStruct((NUM_CORES, NPAD, OUT_FEAT), jnp.float32),
    mesh=plsc.VectorSubcoreMesh(
        core_axis_name="c", subcore_axis_name="s",
        num_cores=NUM_CORES, num_subcores=NUM_SUBCORES,
    ),
    scratch_types=(
        [pltpu.VMEM((CHUNK,), jnp.int32)] * 9
        + [pltpu.VMEM((CHUNK, OUT_FEAT), jnp.float32)] * 3
        + [pltpu.VMEM_SHARED((NPAD, OUT_FEAT), jnp.float32)]
        + [pltpu.SemaphoreType.DMA] * 9
    ),
)(_sc_body)


def _add_body(p_ref, o_ref):
    o_ref[...] = p_ref[0] + p_ref[1]


def _merge_parts(parts):
    return pl.pallas_call(
        _add_body,
        grid=(N // BM,),
        in_specs=[pl.BlockSpec((NUM_CORES, BM, OUT_FEAT), lambda i: (0, i, 0))],
        out_specs=pl.BlockSpec((BM, OUT_FEAT), lambda i: (i, 0)),
        out_shape=jax.ShapeDtypeStruct((N, OUT_FEAT), jnp.float32),
    )(parts)


def kernel(x, edge_index, edge_type, weight, bias):
    h, packed, zeros = _transform_nodes(x, weight, bias, edge_index, edge_type)
    table = h.reshape(NUM_RELS * N, OUT_FEAT)
    parts = _sc_gather_scatter(table, packed, zeros)
    return _merge_parts(parts)
